# baseline (speedup 1.0000x reference)
import functools

import jax
import jax.numpy as jnp
from jax import lax
from jax.experimental import pallas as pl
from jax.experimental.pallas import tpu as pltpu
from jax.experimental.pallas import tpu_sc as plsc

CHUNK = 640
N_CHUNKS = 49
SPAN = 31360
RING = 64


def _lane0(vec):
    return vec[0]


def _gather_sc(idx1, table_t, tail_t, B, D, V, NW, NC):
    mesh = plsc.VectorSubcoreMesh(core_axis_name="c", subcore_axis_name="s")
    n_grp = B // 16
    tail = V % 128
    v_al = V - tail

    @functools.partial(
        pl.kernel,
        mesh=mesh,
        out_type=jax.ShapeDtypeStruct((B * D,), jnp.float32),
        scratch_types=[
            pltpu.VMEM((B + 16,), jnp.int32),
            pltpu.VMEM((B + 16,), jnp.int32),
            pltpu.VMEM((32,), jnp.int32),
            pltpu.VMEM((D, CHUNK), jnp.float32),
            pltpu.VMEM((D, CHUNK), jnp.float32),
            pltpu.VMEM((D, tail), jnp.float32),
            pltpu.VMEM((RING * D,), jnp.float32),
            pltpu.SemaphoreType.DMA,
            pltpu.SemaphoreType.DMA,
        ],
        compiler_params=pltpu.CompilerParams(needs_layout_passes=False),
    )
    def k(table_hbm, tail_hbm, idx_hbm, out_hbm,
          idx_v, mu, tu, chunk_a, chunk_b, tail_v, ring_v, sem, sem2):
        wid = lax.axis_index("s") * NC + lax.axis_index("c")
        ulo = wid * SPAN
        uhi = ulo + N_CHUNKS * CHUNK
        lanes = lax.iota(jnp.int32, 16)
        pltpu.sync_copy(idx_hbm, idx_v.at[pl.ds(0, B)])

        def scan_range(g, cnt):
            v = idx_v[pl.ds(g * 16, 16)]
            msk = jnp.logical_and(ulo <= v, v < uhi)
            packed = ((v - ulo) << 14) | (lanes + g * 16)
            plsc.store_compressed(mu.at[pl.ds(cnt, 16)], packed, mask=msk)
            return cnt + _lane0(plsc.all_reduce_population_count(msk))

        def s_of(c):
            return jnp.minimum(ulo + c * CHUNK, v_al - CHUNK)

        def prefetch(c, buf):
            pltpu.async_copy(table_hbm.at[:, pl.ds(s_of(c), CHUNK)], buf, sem2)

        def wait_chunk(buf):
            pltpu.make_async_copy(
                table_hbm.at[:, pl.ds(0, CHUNK)], buf, sem2
            ).wait()

        prefetch(0, chunk_a)
        prefetch(1, chunk_b)

        n_match = lax.fori_loop(0, n_grp, scan_range, 0, unroll=4)
        mu[pl.ds(n_match, 16)] = jnp.full((16,), -1, jnp.int32)
        mgrp = (n_match + 15) // 16

        OCT_W = 8 * CHUNK
        oct_bounds = []
        ocnt = 0
        for oc in range(8):
            lo = oc * OCT_W

            def build(g, cnt, lo=lo):
                p = mu[pl.ds(g * 16, 16)]
                du = p >> 14
                msk = jnp.logical_and(lo <= du, du < lo + OCT_W)
                plsc.store_compressed(idx_v.at[pl.ds(cnt, 16)], p, mask=msk)
                return cnt + _lane0(plsc.all_reduce_population_count(msk))

            oct_bounds.append(ocnt)
            ocnt = lax.fori_loop(0, mgrp, build, ocnt)
        oct_bounds.append(ocnt)
        idx_v[pl.ds(ocnt, 16)] = jnp.full((16,), -1, jnp.int32)
        starts = jnp.zeros((16,), jnp.int32)
        ends = jnp.zeros((16,), jnp.int32)
        for oc in range(8):
            starts = jnp.where(lax.iota(jnp.int32, 16) == oc, oct_bounds[oc], starts)
            ends = jnp.where(lax.iota(jnp.int32, 16) == oc, oct_bounds[oc + 1], ends)

        def process(s, m, buf, width):
            dlo = s - ulo
            oc1 = dlo // OCT_W
            oc2 = (dlo + width - 1) // OCT_W
            oc_starts = jnp.where(lax.iota(jnp.int32, 16) == oc1, starts, 0)
            oc_ends = jnp.where(lax.iota(jnp.int32, 16) == oc2, ends, 0)
            lo_g = jnp.sum(oc_starts) // 16
            hi_g = (jnp.sum(oc_ends) + 15) // 16

            def scan_chunk(g, mm):
                p = idx_v[pl.ds(g * 16, 16)]
                du = p >> 14
                msk = jnp.logical_and(dlo <= du, du < dlo + width)
                plsc.store_compressed(tu.at[pl.ds(0, 16)], p, mask=msk)
                n_g = _lane0(plsc.all_reduce_population_count(msk))

                def per_match(j, m2):
                    p0 = _lane0(tu[pl.ds(j, 16)])
                    u_loc = (p0 >> 14) - dlo
                    b = p0 & 16383
                    cols = jnp.full((16,), u_loc, jnp.int32)
                    slot = (m2 % RING) * D

                    @pl.when(m2 >= RING)
                    def _drain():
                        pltpu.make_async_copy(
                            out_hbm.at[pl.ds(0, D)],
                            ring_v.at[pl.ds(0, D)],
                            sem,
                        ).wait()

                    for kk in range(D // 16):
                        vals = plsc.load_gather(buf, [lanes + 16 * kk, cols])
                        ring_v[pl.ds(slot + 16 * kk, 16)] = vals
                    pltpu.async_copy(
                        ring_v.at[pl.ds(slot, D)],
                        out_hbm.at[pl.ds(b * D, D)],
                        sem,
                    )
                    return m2 + 1

                return lax.fori_loop(0, n_g, per_match, mm)

            return lax.fori_loop(lo_g, hi_g, scan_chunk, m)

        def pair(g, m):
            wait_chunk(chunk_a)
            m = process(s_of(2 * g), m, chunk_a, CHUNK)
            prefetch(2 * g + 2, chunk_a)
            wait_chunk(chunk_b)
            m = process(s_of(2 * g + 1), m, chunk_b, CHUNK)
            prefetch(2 * g + 3, chunk_b)
            return m

        m_total = lax.fori_loop(0, N_CHUNKS // 2, pair, 0)
        wait_chunk(chunk_a)
        m_total = process(s_of(N_CHUNKS - 1), m_total, chunk_a, CHUNK)
        wait_chunk(chunk_b)
        if tail:
            pltpu.sync_copy(tail_hbm, tail_v)
            m_total = process(v_al, m_total, tail_v, tail)

        def drain_rest(j, _):
            pltpu.make_async_copy(
                out_hbm.at[pl.ds(0, D)], ring_v.at[pl.ds(0, D)], sem
            ).wait()
            return 0

        lax.fori_loop(0, jnp.minimum(m_total, RING), drain_rest, 0)

    return k(table_t, tail_t, idx1)


def kernel(user_idxs, emb_table):
    (B,) = user_idxs.shape
    V, D = emb_table.shape
    info = plsc.get_sparse_core_info()
    NC, NS = info.num_cores, info.num_subcores
    NW = NC * NS
    tail_t = emb_table[V - V % 128:].T
    out1 = _gather_sc(
        user_idxs.astype(jnp.int32), emb_table.T, tail_t, B, D, V, NW, NC
    )
    return out1.reshape(B, D)


# speedup vs baseline: 1.0074x; 1.0074x over previous
import functools

import jax
import jax.numpy as jnp
from jax import lax
from jax.experimental import pallas as pl
from jax.experimental.pallas import tpu as pltpu
from jax.experimental.pallas import tpu_sc as plsc

CHUNK = 640
N_CHUNKS = 49
SPAN = 31360
RING = 64


def _lane0(vec):
    return vec[0]


def _gather_sc(idx1, table_t, tail_t, B, D, V, NW, NC):
    mesh = plsc.VectorSubcoreMesh(core_axis_name="c", subcore_axis_name="s")
    n_grp = B // 16
    tail = V % 128
    v_al = V - tail

    @functools.partial(
        pl.kernel,
        mesh=mesh,
        out_type=jax.ShapeDtypeStruct((B * D,), jnp.float32),
        scratch_types=[
            pltpu.VMEM((B + 16,), jnp.int32),
            pltpu.VMEM((B + 16,), jnp.int32),
            pltpu.VMEM((32,), jnp.int32),
            pltpu.VMEM((D, CHUNK), jnp.float32),
            pltpu.VMEM((D, CHUNK), jnp.float32),
            pltpu.VMEM((D, tail), jnp.float32),
            pltpu.VMEM((RING * D,), jnp.float32),
            pltpu.SemaphoreType.DMA,
            pltpu.SemaphoreType.DMA,
        ],
        compiler_params=pltpu.CompilerParams(needs_layout_passes=False),
    )
    def k(table_hbm, tail_hbm, idx_hbm, out_hbm,
          idx_v, mu, tu, chunk_a, chunk_b, tail_v, ring_v, sem, sem2):
        wid = lax.axis_index("s") * NC + lax.axis_index("c")
        ulo = wid * SPAN
        uhi = ulo + N_CHUNKS * CHUNK
        lanes = lax.iota(jnp.int32, 16)
        pltpu.sync_copy(idx_hbm, idx_v.at[pl.ds(0, B)])

        def scan_range(g, cnt):
            v = idx_v[pl.ds(g * 16, 16)]
            msk = jnp.logical_and(ulo <= v, v < uhi)
            packed = ((v - ulo) << 14) | (lanes + g * 16)
            plsc.store_compressed(mu.at[pl.ds(cnt, 16)], packed, mask=msk)
            return cnt + _lane0(plsc.all_reduce_population_count(msk))

        def s_of(c):
            return jnp.minimum(ulo + c * CHUNK, v_al - CHUNK)

        def prefetch(c, buf):
            pltpu.async_copy(table_hbm.at[:, pl.ds(s_of(c), CHUNK)], buf, sem2)

        def wait_chunk(buf):
            pltpu.make_async_copy(
                table_hbm.at[:, pl.ds(0, CHUNK)], buf, sem2
            ).wait()

        prefetch(0, chunk_a)
        prefetch(1, chunk_b)

        n_match = lax.fori_loop(0, n_grp, scan_range, 0)
        mu[pl.ds(n_match, 16)] = jnp.full((16,), -1, jnp.int32)
        mgrp = (n_match + 15) // 16

        OCT_W = 8 * CHUNK
        oct_bounds = []
        ocnt = 0
        for oc in range(8):
            lo = oc * OCT_W

            def build(g, cnt, lo=lo):
                p = mu[pl.ds(g * 16, 16)]
                du = p >> 14
                msk = jnp.logical_and(lo <= du, du < lo + OCT_W)
                plsc.store_compressed(idx_v.at[pl.ds(cnt, 16)], p, mask=msk)
                return cnt + _lane0(plsc.all_reduce_population_count(msk))

            oct_bounds.append(ocnt)
            ocnt = lax.fori_loop(0, mgrp, build, ocnt)
        oct_bounds.append(ocnt)
        idx_v[pl.ds(ocnt, 16)] = jnp.full((16,), -1, jnp.int32)
        starts = jnp.zeros((16,), jnp.int32)
        ends = jnp.zeros((16,), jnp.int32)
        for oc in range(8):
            starts = jnp.where(lax.iota(jnp.int32, 16) == oc, oct_bounds[oc], starts)
            ends = jnp.where(lax.iota(jnp.int32, 16) == oc, oct_bounds[oc + 1], ends)

        def process(s, m, buf, width):
            dlo = s - ulo
            oc1 = dlo // OCT_W
            oc2 = (dlo + width - 1) // OCT_W
            oc_starts = jnp.where(lax.iota(jnp.int32, 16) == oc1, starts, 0)
            oc_ends = jnp.where(lax.iota(jnp.int32, 16) == oc2, ends, 0)
            lo_g = jnp.sum(oc_starts) // 16
            hi_g = (jnp.sum(oc_ends) + 15) // 16

            def scan_chunk(g, mm):
                p = idx_v[pl.ds(g * 16, 16)]
                du = p >> 14
                msk = jnp.logical_and(dlo <= du, du < dlo + width)
                plsc.store_compressed(tu.at[pl.ds(0, 16)], p, mask=msk)
                n_g = _lane0(plsc.all_reduce_population_count(msk))

                def per_match(j, m2):
                    p0 = _lane0(tu[pl.ds(j, 16)])
                    u_loc = (p0 >> 14) - dlo
                    b = p0 & 16383
                    cols = jnp.full((16,), u_loc, jnp.int32)
                    slot = (m2 % RING) * D

                    @pl.when(m2 >= RING)
                    def _drain():
                        pltpu.make_async_copy(
                            out_hbm.at[pl.ds(0, D)],
                            ring_v.at[pl.ds(0, D)],
                            sem,
                        ).wait()

                    for kk in range(D // 16):
                        vals = plsc.load_gather(buf, [lanes + 16 * kk, cols])
                        ring_v[pl.ds(slot + 16 * kk, 16)] = vals
                    pltpu.async_copy(
                        ring_v.at[pl.ds(slot, D)],
                        out_hbm.at[pl.ds(b * D, D)],
                        sem,
                    )
                    return m2 + 1

                return lax.fori_loop(0, n_g, per_match, mm)

            return lax.fori_loop(lo_g, hi_g, scan_chunk, m)

        def pair(g, m):
            wait_chunk(chunk_a)
            m = process(s_of(2 * g), m, chunk_a, CHUNK)
            prefetch(2 * g + 2, chunk_a)
            wait_chunk(chunk_b)
            m = process(s_of(2 * g + 1), m, chunk_b, CHUNK)
            prefetch(2 * g + 3, chunk_b)
            return m

        m_total = lax.fori_loop(0, N_CHUNKS // 2, pair, 0)
        wait_chunk(chunk_a)
        m_total = process(s_of(N_CHUNKS - 1), m_total, chunk_a, CHUNK)
        wait_chunk(chunk_b)
        if tail:
            pltpu.sync_copy(tail_hbm, tail_v)
            m_total = process(v_al, m_total, tail_v, tail)

        def drain_rest(j, _):
            pltpu.make_async_copy(
                out_hbm.at[pl.ds(0, D)], ring_v.at[pl.ds(0, D)], sem
            ).wait()
            return 0

        lax.fori_loop(0, jnp.minimum(m_total, RING), drain_rest, 0)

    return k(table_t, tail_t, idx1)


def kernel(user_idxs, emb_table):
    (B,) = user_idxs.shape
    V, D = emb_table.shape
    info = plsc.get_sparse_core_info()
    NC, NS = info.num_cores, info.num_subcores
    NW = NC * NS
    tail_t = emb_table[V - V % 128:].T
    out1 = _gather_sc(
        user_idxs.astype(jnp.int32), emb_table.T, tail_t, B, D, V, NW, NC
    )
    return out1.reshape(B, D)
